# baseline (speedup 1.0000x reference)
import functools

import jax
import jax.numpy as jnp
from jax import lax
from jax.experimental import pallas as pl
from jax.experimental.pallas import tpu as pltpu
from jax.experimental.pallas import tpu_sc as plsc

D = 64
EPS = 1e-12
CHUNK = 128
NBUF = 5


@functools.lru_cache(maxsize=None)
def _sc_gather_fn(n_chunks_total: int, n_pairs: int):
    info = plsc.get_sparse_core_info()
    nw = info.num_cores * info.num_subcores
    t = n_chunks_total // nw
    assert t * nw == n_chunks_total and t % NBUF == 0
    n_iter = t // NBUF
    mesh = plsc.VectorSubcoreMesh(core_axis_name="c", subcore_axis_name="s")

    @functools.partial(
        pl.kernel,
        mesh=mesh,
        compiler_params=pltpu.CompilerParams(use_tc_tiling_on_sc=False),
        out_type=jax.ShapeDtypeStruct((n_chunks_total * CHUNK, 2 * D),
                                      jnp.bfloat16),
        scratch_types=(
            [pltpu.VMEM((t, CHUNK), jnp.int32)]
            + [pltpu.VMEM((CHUNK, 2 * D), jnp.bfloat16) for _ in range(NBUF)]
            + [pltpu.SemaphoreType.DMA for _ in range(2 * NBUF)]
        ),
    )
    def gather_kernel(ids_hbm, table_hbm, out_hbm, idx_v, *rest):
        bufs = rest[:NBUF]
        gsem = rest[NBUF:2 * NBUF]
        osem = rest[2 * NBUF:]
        wid = lax.axis_index("s") * info.num_cores + lax.axis_index("c")
        chunk0 = wid * t
        row0 = chunk0 * CHUNK

        pltpu.sync_copy(ids_hbm.at[pl.ds(chunk0, t)], idx_v)

        def start_gather(j, b):
            pltpu.async_copy(table_hbm.at[idx_v.at[j]], bufs[b], gsem[b])

        def wait_gather(j, b):
            pltpu.make_async_copy(table_hbm.at[idx_v.at[j]], bufs[b], gsem[b]).wait()

        def start_store(j, b):
            pltpu.async_copy(bufs[b], out_hbm.at[pl.ds(row0 + j * CHUNK, CHUNK)],
                             osem[b])

        def wait_store(j, b):
            pltpu.make_async_copy(bufs[b],
                                  out_hbm.at[pl.ds(row0 + j * CHUNK, CHUNK)],
                                  osem[b]).wait()

        for b in range(NBUF):
            start_gather(b, b)

        def body(g, carry):
            for b in range(NBUF):
                j = g * NBUF + b
                wait_gather(j, b)
                start_store(j, b)

            @pl.when(g + 1 < n_iter)
            def _():
                for b in range(NBUF):
                    jn = (g + 1) * NBUF + b
                    wait_store(jn - NBUF, b)
                    start_gather(jn, b)

            return carry

        lax.fori_loop(0, n_iter, body, 0)
        for b in range(NBUF):
            wait_store((n_iter - 1) * NBUF + b, b)

    return gather_kernel


TBT = 2048


def _tp_body(x_ref, o_ref):
    x = x_ref[...]
    xs = jnp.concatenate([x[:, :TBT // 2], x[:, TBT // 2:]], axis=0)
    o_ref[...] = xs.T.astype(jnp.bfloat16)


def _transpose_pack(table_t, vocab: int):
    nsb = (vocab + TBT - 1) // TBT
    return pl.pallas_call(
        _tp_body,
        grid=(nsb,),
        in_specs=[pl.BlockSpec((D, TBT), lambda i: (0, i))],
        out_specs=pl.BlockSpec((TBT // 2, 2 * D), lambda i: (i, 0)),
        out_shape=jax.ShapeDtypeStruct((nsb * (TBT // 2), 2 * D), jnp.bfloat16),
    )(table_t)


def _ln_body(x_ref, c_ref, g_ref, b_ref, p_ref, par_ref, o_ref):
    x = x_ref[...].astype(jnp.float32) + c_ref[...]
    p = p_ref[...]
    m = jax.lax.dot(x, p, precision=lax.Precision.DEFAULT)
    sq = jax.lax.dot(x * x, p, precision=lax.Precision.DEFAULT)
    v = sq - m * m
    y = (x - m) * lax.rsqrt(v + EPS) * g_ref[...] + b_ref[...]
    t = y.T
    sel = jnp.where(par_ref[0] == 1, t[D:], t[:D])
    o_ref[...] = sel.reshape(1, D, sel.shape[-1])


def _layernorm_q(lines, common2, gamma2, beta2, pmat, par3, s1: int, s0: int,
                 bs: int):
    nblk = s0 // bs
    return pl.pallas_call(
        _ln_body,
        grid=(s1, nblk),
        in_specs=[
            pl.BlockSpec((bs, 2 * D), lambda q, i: (q * nblk + i, 0)),
            pl.BlockSpec((1, 2 * D), lambda q, i: (0, 0)),
            pl.BlockSpec((1, 2 * D), lambda q, i: (0, 0)),
            pl.BlockSpec((1, 2 * D), lambda q, i: (0, 0)),
            pl.BlockSpec((2 * D, 2 * D), lambda q, i: (0, 0)),
            pl.BlockSpec((1, 1, bs), lambda q, i: (q, 0, i)),
        ],
        out_specs=pl.BlockSpec((1, D, bs), lambda q, i: (q, 0, i)),
        out_shape=jax.ShapeDtypeStruct((s1, D, s0), jnp.float32),
    )(lines, common2, gamma2, beta2, pmat, par3)


def kernel(input_ids, table, common, gamma, beta):
    s0, s1 = input_ids.shape
    b = s0 * s1
    idsp = input_ids.transpose(1, 0).astype(jnp.int32)
    half = TBT // 2
    sh = TBT.bit_length() - 1
    pidx = (((idsp >> sh) * half) | (idsp & (half - 1))).reshape(b // CHUNK, CHUNK)
    par3 = ((idsp >> (sh - 1)) & 1).reshape(s1, 1, s0)
    tpack = _transpose_pack(table.T, table.shape[0])
    lines = _sc_gather_fn(b // CHUNK, tpack.shape[0])(pidx, tpack)
    dup = lambda a: jnp.concatenate([a.reshape(1, D), a.reshape(1, D)], axis=1)
    lane = jax.lax.broadcasted_iota(jnp.int32, (2 * D, 2 * D), 0)
    lane_t = jax.lax.broadcasted_iota(jnp.int32, (2 * D, 2 * D), 1)
    pmat = jnp.where((lane // D) == (lane_t // D), 1.0 / D, 0.0).astype(jnp.float32)
    z3 = _layernorm_q(lines, dup(common), dup(gamma), dup(beta), pmat, par3,
                      s1, s0, bs=4096)
    return z3.transpose(2, 0, 1)


# speedup vs baseline: 2.1131x; 2.1131x over previous
import functools

import jax
import jax.numpy as jnp
from jax import lax
from jax.experimental import pallas as pl
from jax.experimental.pallas import tpu as pltpu
from jax.experimental.pallas import tpu_sc as plsc

D = 64
EPS = 1e-12
CHUNK = 128
NBUF = 5


@functools.lru_cache(maxsize=None)
def _sc_gather_fn(n_chunks_total: int, n_pairs: int):
    info = plsc.get_sparse_core_info()
    nw = info.num_cores * info.num_subcores
    t = n_chunks_total // nw
    assert t * nw == n_chunks_total and t % NBUF == 0
    n_iter = t // NBUF
    mesh = plsc.VectorSubcoreMesh(core_axis_name="c", subcore_axis_name="s")

    @functools.partial(
        pl.kernel,
        mesh=mesh,
        compiler_params=pltpu.CompilerParams(use_tc_tiling_on_sc=False),
        out_type=jax.ShapeDtypeStruct((n_chunks_total * CHUNK, 2 * D),
                                      jnp.float32),
        scratch_types=(
            [pltpu.VMEM((t, CHUNK), jnp.int32)]
            + [pltpu.VMEM((CHUNK, 2 * D), jnp.float32) for _ in range(NBUF)]
            + [pltpu.SemaphoreType.DMA for _ in range(2 * NBUF)]
        ),
    )
    def gather_kernel(ids_hbm, table_hbm, out_hbm, idx_v, *rest):
        bufs = rest[:NBUF]
        gsem = rest[NBUF:2 * NBUF]
        osem = rest[2 * NBUF:]
        wid = lax.axis_index("s") * info.num_cores + lax.axis_index("c")
        chunk0 = wid * t
        row0 = chunk0 * CHUNK

        pltpu.sync_copy(ids_hbm.at[pl.ds(chunk0, t)], idx_v)

        def start_gather(j, b):
            pltpu.async_copy(table_hbm.at[idx_v.at[j]], bufs[b], gsem[b])

        def wait_gather(j, b):
            pltpu.make_async_copy(table_hbm.at[idx_v.at[j]], bufs[b], gsem[b]).wait()

        def start_store(j, b):
            pltpu.async_copy(bufs[b], out_hbm.at[pl.ds(row0 + j * CHUNK, CHUNK)],
                             osem[b])

        def wait_store(j, b):
            pltpu.make_async_copy(bufs[b],
                                  out_hbm.at[pl.ds(row0 + j * CHUNK, CHUNK)],
                                  osem[b]).wait()

        for b in range(NBUF):
            start_gather(b, b)

        def body(g, carry):
            for b in range(NBUF):
                j = g * NBUF + b
                wait_gather(j, b)
                start_store(j, b)

            @pl.when(g + 1 < n_iter)
            def _():
                for b in range(NBUF):
                    jn = (g + 1) * NBUF + b
                    wait_store(jn - NBUF, b)
                    start_gather(jn, b)

            return carry

        lax.fori_loop(0, n_iter, body, 0)
        for b in range(NBUF):
            wait_store((n_iter - 1) * NBUF + b, b)

    return gather_kernel


TBT = 2048


def _tp_body(x_ref, o_ref):
    x = x_ref[...]
    xs = jnp.concatenate([x[:, :TBT // 2], x[:, TBT // 2:]], axis=0)
    o_ref[...] = xs.T


def _transpose_pack(table_t, vocab: int):
    nsb = (vocab + TBT - 1) // TBT
    return pl.pallas_call(
        _tp_body,
        grid=(nsb,),
        in_specs=[pl.BlockSpec((D, TBT), lambda i: (0, i))],
        out_specs=pl.BlockSpec((TBT // 2, 2 * D), lambda i: (i, 0)),
        out_shape=jax.ShapeDtypeStruct((nsb * (TBT // 2), 2 * D), jnp.float32),
    )(table_t)


def _ln_body(x_ref, c_ref, g_ref, b_ref, p_ref, par_ref, o_ref):
    x = x_ref[...] + c_ref[...]
    p = p_ref[...]
    m = jax.lax.dot(x, p, precision=lax.Precision.DEFAULT)
    sq = jax.lax.dot(x * x, p, precision=lax.Precision.DEFAULT)
    v = sq - m * m
    y = (x - m) * lax.rsqrt(v + EPS) * g_ref[...] + b_ref[...]
    t = y.T
    sel = jnp.where(par_ref[0] == 1, t[D:], t[:D])
    o_ref[...] = sel.reshape(1, D, sel.shape[-1])


def _layernorm_q(lines, common2, gamma2, beta2, pmat, par3, s1: int, s0: int,
                 bs: int):
    nblk = s0 // bs
    return pl.pallas_call(
        _ln_body,
        grid=(s1, nblk),
        in_specs=[
            pl.BlockSpec((bs, 2 * D), lambda q, i: (q * nblk + i, 0)),
            pl.BlockSpec((1, 2 * D), lambda q, i: (0, 0)),
            pl.BlockSpec((1, 2 * D), lambda q, i: (0, 0)),
            pl.BlockSpec((1, 2 * D), lambda q, i: (0, 0)),
            pl.BlockSpec((2 * D, 2 * D), lambda q, i: (0, 0)),
            pl.BlockSpec((1, 1, bs), lambda q, i: (q, 0, i)),
        ],
        out_specs=pl.BlockSpec((1, D, bs), lambda q, i: (q, 0, i)),
        out_shape=jax.ShapeDtypeStruct((s1, D, s0), jnp.float32),
    )(lines, common2, gamma2, beta2, pmat, par3)


def kernel(input_ids, table, common, gamma, beta):
    s0, s1 = input_ids.shape
    b = s0 * s1
    idsp = input_ids.transpose(1, 0).astype(jnp.int32)
    half = TBT // 2
    sh = TBT.bit_length() - 1
    pidx = (((idsp >> sh) * half) | (idsp & (half - 1))).reshape(b // CHUNK, CHUNK)
    par3 = ((idsp >> (sh - 1)) & 1).reshape(s1, 1, s0)
    tpack = _transpose_pack(table.T, table.shape[0])
    lines = _sc_gather_fn(b // CHUNK, tpack.shape[0])(pidx, tpack)
    dup = lambda a: jnp.concatenate([a.reshape(1, D), a.reshape(1, D)], axis=1)
    lane = jax.lax.broadcasted_iota(jnp.int32, (2 * D, 2 * D), 0)
    lane_t = jax.lax.broadcasted_iota(jnp.int32, (2 * D, 2 * D), 1)
    pmat = jnp.where((lane // D) == (lane_t // D), 1.0 / D, 0.0).astype(jnp.float32)
    z3 = _layernorm_q(lines, dup(common), dup(gamma), dup(beta), pmat, par3,
                      s1, s0, bs=4096)
    return z3.transpose(2, 0, 1)


# speedup vs baseline: 2.6439x; 1.2512x over previous
import functools

import jax
import jax.numpy as jnp
from jax import lax
from jax.experimental import pallas as pl
from jax.experimental.pallas import tpu as pltpu
from jax.experimental.pallas import tpu_sc as plsc

D = 64
EPS = 1e-12
CHUNK = 128
NBUF = 5


@functools.lru_cache(maxsize=None)
def _sc_gather_fn(n_chunks_total: int, n_pairs: int):
    info = plsc.get_sparse_core_info()
    nw = info.num_cores * info.num_subcores
    t = n_chunks_total // nw
    assert t * nw == n_chunks_total and t % NBUF == 0
    n_iter = t // NBUF
    mesh = plsc.VectorSubcoreMesh(core_axis_name="c", subcore_axis_name="s")

    @functools.partial(
        pl.kernel,
        mesh=mesh,
        compiler_params=pltpu.CompilerParams(use_tc_tiling_on_sc=False),
        out_type=jax.ShapeDtypeStruct((n_chunks_total * CHUNK, 2 * D),
                                      jnp.float32),
        scratch_types=(
            [pltpu.VMEM((t, CHUNK), jnp.int32)]
            + [pltpu.VMEM((CHUNK, 2 * D), jnp.float32) for _ in range(NBUF)]
            + [pltpu.SemaphoreType.DMA for _ in range(2 * NBUF)]
        ),
    )
    def gather_kernel(ids_hbm, table_hbm, out_hbm, idx_v, *rest):
        bufs = rest[:NBUF]
        gsem = rest[NBUF:2 * NBUF]
        osem = rest[2 * NBUF:]
        wid = lax.axis_index("s") * info.num_cores + lax.axis_index("c")
        chunk0 = wid * t
        row0 = chunk0 * CHUNK

        pltpu.sync_copy(ids_hbm.at[pl.ds(chunk0, t)], idx_v)

        def start_gather(j, b):
            pltpu.async_copy(table_hbm.at[idx_v.at[j]], bufs[b], gsem[b])

        def wait_gather(j, b):
            pltpu.make_async_copy(table_hbm.at[idx_v.at[j]], bufs[b], gsem[b]).wait()

        def start_store(j, b):
            pltpu.async_copy(bufs[b], out_hbm.at[pl.ds(row0 + j * CHUNK, CHUNK)],
                             osem[b])

        def wait_store(j, b):
            pltpu.make_async_copy(bufs[b],
                                  out_hbm.at[pl.ds(row0 + j * CHUNK, CHUNK)],
                                  osem[b]).wait()

        for b in range(NBUF):
            start_gather(b, b)

        def body(g, carry):
            for b in range(NBUF):
                j = g * NBUF + b
                wait_gather(j, b)
                start_store(j, b)

            @pl.when(g + 1 < n_iter)
            def _():
                for b in range(NBUF):
                    jn = (g + 1) * NBUF + b
                    wait_store(jn - NBUF, b)
                    start_gather(jn, b)

            return carry

        lax.fori_loop(0, n_iter, body, 0)
        for b in range(NBUF):
            wait_store((n_iter - 1) * NBUF + b, b)

    return gather_kernel


TBT = 8192


def _tp_body(x_ref, o_ref):
    x = x_ref[...]
    xs = jnp.concatenate([x[:, :TBT // 2], x[:, TBT // 2:]], axis=0)
    o_ref[...] = xs.T


def _transpose_pack(table_t, vocab: int):
    nsb = (vocab + TBT - 1) // TBT
    return pl.pallas_call(
        _tp_body,
        grid=(nsb,),
        in_specs=[pl.BlockSpec((D, TBT), lambda i: (0, i))],
        out_specs=pl.BlockSpec((TBT // 2, 2 * D), lambda i: (i, 0)),
        out_shape=jax.ShapeDtypeStruct((nsb * (TBT // 2), 2 * D), jnp.float32),
    )(table_t)


def _ln_body(x_ref, c_ref, g_ref, b_ref, p_ref, par_ref, o_ref):
    x = x_ref[...] + c_ref[...]
    p = p_ref[...]
    m = jax.lax.dot(x, p, precision=lax.Precision.DEFAULT)
    sq = jax.lax.dot(x * x, p, precision=lax.Precision.DEFAULT)
    v = sq - m * m
    y = (x - m) * lax.rsqrt(v + EPS) * g_ref[...] + b_ref[...]
    t = y.T
    sel = jnp.where(par_ref[0] == 1, t[D:], t[:D])
    o_ref[...] = sel.reshape(1, D, sel.shape[-1])


def _layernorm_q(lines, common2, gamma2, beta2, pmat, par3, s1: int, s0: int,
                 bs: int):
    nblk = s0 // bs
    return pl.pallas_call(
        _ln_body,
        grid=(s1, nblk),
        in_specs=[
            pl.BlockSpec((bs, 2 * D), lambda q, i: (q * nblk + i, 0)),
            pl.BlockSpec((1, 2 * D), lambda q, i: (0, 0)),
            pl.BlockSpec((1, 2 * D), lambda q, i: (0, 0)),
            pl.BlockSpec((1, 2 * D), lambda q, i: (0, 0)),
            pl.BlockSpec((2 * D, 2 * D), lambda q, i: (0, 0)),
            pl.BlockSpec((1, 1, bs), lambda q, i: (q, 0, i)),
        ],
        out_specs=pl.BlockSpec((1, D, bs), lambda q, i: (q, 0, i)),
        out_shape=jax.ShapeDtypeStruct((s1, D, s0), jnp.float32),
    )(lines, common2, gamma2, beta2, pmat, par3)


def kernel(input_ids, table, common, gamma, beta):
    s0, s1 = input_ids.shape
    b = s0 * s1
    idsp = input_ids.transpose(1, 0).astype(jnp.int32)
    half = TBT // 2
    sh = TBT.bit_length() - 1
    pidx = (((idsp >> sh) * half) | (idsp & (half - 1))).reshape(b // CHUNK, CHUNK)
    par3 = ((idsp >> (sh - 1)) & 1).reshape(s1, 1, s0)
    tpack = _transpose_pack(table.T, table.shape[0])
    lines = _sc_gather_fn(b // CHUNK, tpack.shape[0])(pidx, tpack)
    dup = lambda a: jnp.concatenate([a.reshape(1, D), a.reshape(1, D)], axis=1)
    lane = jax.lax.broadcasted_iota(jnp.int32, (2 * D, 2 * D), 0)
    lane_t = jax.lax.broadcasted_iota(jnp.int32, (2 * D, 2 * D), 1)
    pmat = jnp.where((lane // D) == (lane_t // D), 1.0 / D, 0.0).astype(jnp.float32)
    z3 = _layernorm_q(lines, dup(common), dup(gamma), dup(beta), pmat, par3,
                      s1, s0, bs=8192)
    return z3.transpose(2, 0, 1)


# speedup vs baseline: 2.8778x; 1.0885x over previous
import functools

import jax
import jax.numpy as jnp
from jax import lax
from jax.experimental import pallas as pl
from jax.experimental.pallas import tpu as pltpu
from jax.experimental.pallas import tpu_sc as plsc

D = 64
EPS = 1e-12
CHUNK = 128
NBUF = 5


@functools.lru_cache(maxsize=None)
def _sc_gather_fn(n_chunks_total: int, n_pairs: int):
    info = plsc.get_sparse_core_info()
    nw = info.num_cores * info.num_subcores
    t = n_chunks_total // nw
    assert t * nw == n_chunks_total and t % NBUF == 0
    n_iter = t // NBUF
    mesh = plsc.VectorSubcoreMesh(core_axis_name="c", subcore_axis_name="s")

    @functools.partial(
        pl.kernel,
        mesh=mesh,
        compiler_params=pltpu.CompilerParams(use_tc_tiling_on_sc=False),
        out_type=jax.ShapeDtypeStruct((n_chunks_total * CHUNK, 2 * D),
                                      jnp.float32),
        scratch_types=(
            [pltpu.VMEM((t, CHUNK), jnp.int32)]
            + [pltpu.VMEM((CHUNK, 2 * D), jnp.float32) for _ in range(NBUF)]
            + [pltpu.SemaphoreType.DMA for _ in range(2 * NBUF)]
        ),
    )
    def gather_kernel(ids_hbm, table_hbm, out_hbm, idx_v, *rest):
        bufs = rest[:NBUF]
        gsem = rest[NBUF:2 * NBUF]
        osem = rest[2 * NBUF:]
        wid = lax.axis_index("s") * info.num_cores + lax.axis_index("c")
        chunk0 = wid * t
        row0 = chunk0 * CHUNK

        pltpu.sync_copy(ids_hbm.at[pl.ds(chunk0, t)], idx_v)

        def start_gather(j, b):
            pltpu.async_copy(table_hbm.at[idx_v.at[j]], bufs[b], gsem[b])

        def wait_gather(j, b):
            pltpu.make_async_copy(table_hbm.at[idx_v.at[j]], bufs[b], gsem[b]).wait()

        def start_store(j, b):
            pltpu.async_copy(bufs[b], out_hbm.at[pl.ds(row0 + j * CHUNK, CHUNK)],
                             osem[b])

        def wait_store(j, b):
            pltpu.make_async_copy(bufs[b],
                                  out_hbm.at[pl.ds(row0 + j * CHUNK, CHUNK)],
                                  osem[b]).wait()

        for b in range(NBUF):
            start_gather(b, b)

        def body(g, carry):
            for b in range(NBUF):
                j = g * NBUF + b
                wait_gather(j, b)
                start_store(j, b)

            @pl.when(g + 1 < n_iter)
            def _():
                for b in range(NBUF):
                    jn = (g + 1) * NBUF + b
                    wait_store(jn - NBUF, b)
                    start_gather(jn, b)

            return carry

        lax.fori_loop(0, n_iter, body, 0)
        for b in range(NBUF):
            wait_store((n_iter - 1) * NBUF + b, b)

    return gather_kernel


TBT = 16384


def _tp_body(x_ref, o_ref):
    x = x_ref[...]
    xs = jnp.concatenate([x[:, :TBT // 2], x[:, TBT // 2:]], axis=0)
    o_ref[...] = xs.T


def _transpose_pack(table_t, vocab: int):
    nsb = (vocab + TBT - 1) // TBT
    return pl.pallas_call(
        _tp_body,
        grid=(nsb,),
        in_specs=[pl.BlockSpec((D, TBT), lambda i: (0, i))],
        out_specs=pl.BlockSpec((TBT // 2, 2 * D), lambda i: (i, 0)),
        out_shape=jax.ShapeDtypeStruct((nsb * (TBT // 2), 2 * D), jnp.float32),
    )(table_t)


def _ln_body(x_ref, c_ref, g_ref, b_ref, p_ref, par_ref, o_ref):
    x = x_ref[...] + c_ref[...]
    p = p_ref[...]
    m = jax.lax.dot(x, p, precision=lax.Precision.DEFAULT)
    sq = jax.lax.dot(x * x, p, precision=lax.Precision.DEFAULT)
    v = sq - m * m
    y = (x - m) * lax.rsqrt(v + EPS) * g_ref[...] + b_ref[...]
    t = y.T
    sel = jnp.where(par_ref[0] == 1, t[D:], t[:D])
    o_ref[...] = sel.reshape(1, D, sel.shape[-1])


def _layernorm_q(lines, common2, gamma2, beta2, pmat, par3, s1: int, s0: int,
                 bs: int):
    nblk = s0 // bs
    return pl.pallas_call(
        _ln_body,
        grid=(s1, nblk),
        in_specs=[
            pl.BlockSpec((bs, 2 * D), lambda q, i: (q * nblk + i, 0)),
            pl.BlockSpec((1, 2 * D), lambda q, i: (0, 0)),
            pl.BlockSpec((1, 2 * D), lambda q, i: (0, 0)),
            pl.BlockSpec((1, 2 * D), lambda q, i: (0, 0)),
            pl.BlockSpec((2 * D, 2 * D), lambda q, i: (0, 0)),
            pl.BlockSpec((1, 1, bs), lambda q, i: (q, 0, i)),
        ],
        out_specs=pl.BlockSpec((1, D, bs), lambda q, i: (q, 0, i)),
        out_shape=jax.ShapeDtypeStruct((s1, D, s0), jnp.float32),
    )(lines, common2, gamma2, beta2, pmat, par3)


def kernel(input_ids, table, common, gamma, beta):
    s0, s1 = input_ids.shape
    b = s0 * s1
    idsp = input_ids.transpose(1, 0).astype(jnp.int32)
    half = TBT // 2
    sh = TBT.bit_length() - 1
    pidx = (((idsp >> sh) * half) | (idsp & (half - 1))).reshape(b // CHUNK, CHUNK)
    par3 = ((idsp >> (sh - 1)) & 1).reshape(s1, 1, s0)
    tpack = _transpose_pack(table.T, table.shape[0])
    lines = _sc_gather_fn(b // CHUNK, tpack.shape[0])(pidx, tpack)
    dup = lambda a: jnp.concatenate([a.reshape(1, D), a.reshape(1, D)], axis=1)
    lane = jax.lax.broadcasted_iota(jnp.int32, (2 * D, 2 * D), 0)
    lane_t = jax.lax.broadcasted_iota(jnp.int32, (2 * D, 2 * D), 1)
    pmat = jnp.where((lane // D) == (lane_t // D), 1.0 / D, 0.0).astype(jnp.float32)
    z3 = _layernorm_q(lines, dup(common), dup(gamma), dup(beta), pmat, par3,
                      s1, s0, bs=16384)
    return z3.transpose(2, 0, 1)


# speedup vs baseline: 2.9112x; 1.0116x over previous
import functools

import jax
import jax.numpy as jnp
from jax import lax
from jax.experimental import pallas as pl
from jax.experimental.pallas import tpu as pltpu
from jax.experimental.pallas import tpu_sc as plsc

D = 64
EPS = 1e-12
CHUNK = 128
NBUF = 5


@functools.lru_cache(maxsize=None)
def _sc_gather_fn(n_chunks_total: int, n_pairs: int):
    info = plsc.get_sparse_core_info()
    nw = info.num_cores * info.num_subcores
    t = n_chunks_total // nw
    assert t * nw == n_chunks_total and t % NBUF == 0
    n_iter = t // NBUF
    mesh = plsc.VectorSubcoreMesh(core_axis_name="c", subcore_axis_name="s")

    @functools.partial(
        pl.kernel,
        mesh=mesh,
        compiler_params=pltpu.CompilerParams(use_tc_tiling_on_sc=False),
        out_type=jax.ShapeDtypeStruct((n_chunks_total * CHUNK, 2 * D),
                                      jnp.float32),
        scratch_types=(
            [pltpu.VMEM((t, CHUNK), jnp.int32)]
            + [pltpu.VMEM((CHUNK, 2 * D), jnp.float32) for _ in range(NBUF)]
            + [pltpu.SemaphoreType.DMA for _ in range(2 * NBUF)]
        ),
    )
    def gather_kernel(ids_hbm, table_hbm, out_hbm, idx_v, *rest):
        bufs = rest[:NBUF]
        gsem = rest[NBUF:2 * NBUF]
        osem = rest[2 * NBUF:]
        wid = lax.axis_index("s") * info.num_cores + lax.axis_index("c")
        chunk0 = wid * t
        row0 = chunk0 * CHUNK

        pltpu.sync_copy(ids_hbm.at[pl.ds(chunk0, t)], idx_v)

        def start_gather(j, b):
            pltpu.async_copy(table_hbm.at[idx_v.at[j]], bufs[b], gsem[b])

        def wait_gather(j, b):
            pltpu.make_async_copy(table_hbm.at[idx_v.at[j]], bufs[b], gsem[b]).wait()

        def start_store(j, b):
            pltpu.async_copy(bufs[b], out_hbm.at[pl.ds(row0 + j * CHUNK, CHUNK)],
                             osem[b])

        def wait_store(j, b):
            pltpu.make_async_copy(bufs[b],
                                  out_hbm.at[pl.ds(row0 + j * CHUNK, CHUNK)],
                                  osem[b]).wait()

        for b in range(NBUF):
            start_gather(b, b)

        def body(g, carry):
            for b in range(NBUF):
                j = g * NBUF + b
                wait_gather(j, b)
                start_store(j, b)

            @pl.when(g + 1 < n_iter)
            def _():
                for b in range(NBUF):
                    jn = (g + 1) * NBUF + b
                    wait_store(jn - NBUF, b)
                    start_gather(jn, b)

            return carry

        lax.fori_loop(0, n_iter, body, 0)
        for b in range(NBUF):
            wait_store((n_iter - 1) * NBUF + b, b)

    return gather_kernel


TBT = 16384


def _tp_body(x_ref, o_ref):
    x = x_ref[...]
    xs = jnp.concatenate([x[:, :TBT // 2], x[:, TBT // 2:]], axis=0)
    o_ref[...] = xs.T


def _transpose_pack(table_t, vocab: int):
    nsb = (vocab + TBT - 1) // TBT
    return pl.pallas_call(
        _tp_body,
        grid=(nsb,),
        in_specs=[pl.BlockSpec((D, TBT), lambda i: (0, i))],
        out_specs=pl.BlockSpec((TBT // 2, 2 * D), lambda i: (i, 0)),
        out_shape=jax.ShapeDtypeStruct((nsb * (TBT // 2), 2 * D), jnp.float32),
    )(table_t)


def _ln_body(x_ref, c_ref, g_ref, b_ref, p_ref, par_ref, o_ref):
    x = x_ref[...] + c_ref[...]
    p = p_ref[...]
    m = jax.lax.dot(x, p, precision=lax.Precision.DEFAULT)
    sq = jax.lax.dot(x * x, p, precision=lax.Precision.DEFAULT)
    v = sq - m * m
    y = (x - m) * lax.rsqrt(v + EPS) * g_ref[...] + b_ref[...]
    t = y.T
    sel = jnp.where(par_ref[0] == 1, t[D:], t[:D])
    o_ref[...] = sel.reshape(1, D, sel.shape[-1])


def _ln_body_alias(z_ref, x_ref, c_ref, g_ref, b_ref, p_ref, par_ref, o_ref):
    del z_ref
    _ln_body(x_ref, c_ref, g_ref, b_ref, p_ref, par_ref, o_ref)


def _layernorm_q(lines, common2, gamma2, beta2, pmat, par3, s1: int, s0: int,
                 bs: int, q0: int, nq: int, prev=None):
    nblk = s0 // bs
    data_specs = [
        pl.BlockSpec((bs, 2 * D), lambda q, i: (q * nblk + i, 0)),
        pl.BlockSpec((1, 2 * D), lambda q, i: (0, 0)),
        pl.BlockSpec((1, 2 * D), lambda q, i: (0, 0)),
        pl.BlockSpec((1, 2 * D), lambda q, i: (0, 0)),
        pl.BlockSpec((2 * D, 2 * D), lambda q, i: (0, 0)),
        pl.BlockSpec((1, 1, bs), lambda q, i: (q, 0, i)),
    ]
    out_spec = pl.BlockSpec((1, D, bs), lambda q, i: (q + q0, 0, i))
    out_shape = jax.ShapeDtypeStruct((s1, D, s0), jnp.float32)
    if prev is None:
        return pl.pallas_call(
            _ln_body, grid=(nq, nblk), in_specs=data_specs,
            out_specs=out_spec, out_shape=out_shape,
        )(lines, common2, gamma2, beta2, pmat, par3)
    return pl.pallas_call(
        _ln_body_alias, grid=(nq, nblk),
        in_specs=[pl.BlockSpec(memory_space=pl.ANY)] + data_specs,
        out_specs=out_spec, out_shape=out_shape,
        input_output_aliases={0: 0},
    )(prev, lines, common2, gamma2, beta2, pmat, par3)


def kernel(input_ids, table, common, gamma, beta):
    s0, s1 = input_ids.shape
    b = s0 * s1
    idsp = input_ids.transpose(1, 0).astype(jnp.int32)
    half = TBT // 2
    sh = TBT.bit_length() - 1
    pidx = ((idsp >> sh) * half) | (idsp & (half - 1))
    par3 = ((idsp >> (sh - 1)) & 1).reshape(s1, 1, s0)
    tpack = _transpose_pack(table.T, table.shape[0])
    dup = lambda a: jnp.concatenate([a.reshape(1, D), a.reshape(1, D)], axis=1)
    lane = jax.lax.broadcasted_iota(jnp.int32, (2 * D, 2 * D), 0)
    lane_t = jax.lax.broadcasted_iota(jnp.int32, (2 * D, 2 * D), 1)
    pmat = jnp.where((lane // D) == (lane_t // D), 1.0 / D, 0.0).astype(jnp.float32)
    qh = s1 // 2
    gf = _sc_gather_fn(qh * s0 // CHUNK, tpack.shape[0])
    args = (dup(common), dup(gamma), dup(beta), pmat)
    lines_a = gf(pidx[:qh].reshape(-1, CHUNK), tpack)
    lines_b = gf(pidx[qh:].reshape(-1, CHUNK), tpack)
    z3 = _layernorm_q(lines_a, *args, par3[:qh], s1, s0, bs=16384,
                      q0=0, nq=qh)
    z3 = _layernorm_q(lines_b, *args, par3[qh:], s1, s0, bs=16384,
                      q0=qh, nq=qh, prev=z3)
    return z3.transpose(2, 0, 1)
